# baseline (speedup 1.0000x reference)
import functools

import jax
import jax.numpy as jnp
from jax import lax
from jax.experimental import pallas as pl
from jax.experimental.pallas import tpu as pltpu
from jax.experimental.pallas import tpu_sc as plsc

_NC = 2
_NS = 16
_NW = _NC * _NS
_NB = 7
_LA = 4


@functools.cache
def _build(V, D, B, H):
    n_b = B // _NW
    n_loop = ((H - _NB - _LA) // _NB) * _NB
    n_epi = H - _NB - n_loop
    mesh = plsc.VectorSubcoreMesh(core_axis_name="c", subcore_axis_name="s")

    @functools.partial(
        pl.kernel,
        out_type=jax.ShapeDtypeStruct((H, B, D), jnp.float32),
        mesh=mesh,
        scratch_types=[
            pltpu.VMEM((H, n_b), jnp.int32),
            [pltpu.VMEM((n_b, D), jnp.float32) for _ in range(_NB)],
            [pltpu.SemaphoreType.DMA for _ in range(_NB)],
            [pltpu.SemaphoreType.DMA for _ in range(_NB)],
        ],
    )
    def k(idx_hbm, table_hbm, out_hbm, idx_v, bufs, gsems, wsems):
        wid = lax.axis_index("s") * _NC + lax.axis_index("c")
        base = wid * n_b

        def gather(c, b):
            pltpu.async_copy(table_hbm.at[idx_v.at[c]], bufs[b], gsems[b])

        def wait_gather(b):
            pltpu.make_async_copy(
                table_hbm.at[pl.ds(0, n_b)], bufs[b], gsems[b]).wait()

        def write(c, b):
            pltpu.async_copy(
                bufs[b], out_hbm.at[c, pl.ds(base, n_b)], wsems[b])

        def wait_write(b):
            pltpu.make_async_copy(
                bufs[b], out_hbm.at[0, pl.ds(base, n_b)], wsems[b]).wait()

        def step(c, b, refill_c, need_wwait):
            wait_gather(b)
            write(c, b)
            if refill_c is not None:
                b2 = (b + _LA) % _NB
                if need_wwait:
                    wait_write(b2)
                gather(refill_c, b2)

        pltpu.sync_copy(idx_hbm.at[pl.ds(0, H), pl.ds(base, n_b)], idx_v)
        for c in range(_LA):
            gather(c, c % _NB)
        for c in range(_NB):
            step(c, c % _NB, c + _LA, c >= _NB - _LA)

        @pl.loop(_NB, _NB + n_loop, step=_NB)
        def _(i):
            for b in range(_NB):
                step(i + b, b, i + b + _LA, True)

        for e in range(n_epi):
            c = _NB + n_loop + e
            rc = c + _LA
            step(c, c % _NB, rc if rc < H else None, True)
        for c in range(H - _NB, H):
            wait_write(c % _NB)

    return k


def kernel(seqs, table):
    B, H = seqs.shape
    V, D = table.shape
    assert B % (_NW * 8) == 0
    assert H >= _NB + _LA
    idx_t = seqs.astype(jnp.int32).T
    out_t = _build(V, D, B, H)(idx_t, table)
    return jnp.transpose(out_t, (1, 0, 2))


# speedup vs baseline: 1.0038x; 1.0038x over previous
import functools

import jax
import jax.numpy as jnp
from jax import lax
from jax.experimental import pallas as pl
from jax.experimental.pallas import tpu as pltpu
from jax.experimental.pallas import tpu_sc as plsc

_NC = 2
_NS = 16
_NW = _NC * _NS
_NB = 6
_LA = 3


@functools.cache
def _build(V, D, B, H):
    n_b = B // _NW
    n_loop = ((H - _NB - _LA) // _NB) * _NB
    n_epi = H - _NB - n_loop
    mesh = plsc.VectorSubcoreMesh(core_axis_name="c", subcore_axis_name="s")

    @functools.partial(
        pl.kernel,
        out_type=jax.ShapeDtypeStruct((H, B, D), jnp.float32),
        mesh=mesh,
        scratch_types=[
            pltpu.VMEM((H, n_b), jnp.int32),
            [pltpu.VMEM((n_b, D), jnp.float32) for _ in range(_NB)],
            [pltpu.SemaphoreType.DMA for _ in range(_NB)],
            [pltpu.SemaphoreType.DMA for _ in range(_NB)],
            pltpu.SemaphoreType.DMA,
        ],
    )
    def k(idx_hbm, table_hbm, out_hbm, idx_v, bufs, gsems, wsems, isem):
        wid = lax.axis_index("s") * _NC + lax.axis_index("c")
        base = wid * n_b

        def gather(c, b):
            pltpu.async_copy(table_hbm.at[idx_v.at[c]], bufs[b], gsems[b])

        def wait_gather(b):
            pltpu.make_async_copy(
                table_hbm.at[pl.ds(0, n_b)], bufs[b], gsems[b]).wait()

        def write(c, b):
            pltpu.async_copy(
                bufs[b], out_hbm.at[c, pl.ds(base, n_b)], wsems[b])

        def wait_write(b):
            pltpu.make_async_copy(
                bufs[b], out_hbm.at[0, pl.ds(base, n_b)], wsems[b]).wait()

        def step(c, b, refill_c, need_wwait):
            wait_gather(b)
            write(c, b)
            if refill_c is not None:
                b2 = (b + _LA) % _NB
                if need_wwait:
                    wait_write(b2)
                gather(refill_c, b2)

        pltpu.sync_copy(
            idx_hbm.at[pl.ds(0, 8), pl.ds(base, n_b)], idx_v.at[pl.ds(0, 8)])
        rest = pltpu.async_copy(
            idx_hbm.at[pl.ds(8, H - 8), pl.ds(base, n_b)],
            idx_v.at[pl.ds(8, H - 8)], isem)
        for c in range(_LA):
            gather(c, c % _NB)
        rest.wait()
        for c in range(_NB):
            step(c, c % _NB, c + _LA, c >= _NB - _LA)

        @pl.loop(_NB, _NB + n_loop, step=_NB)
        def _(i):
            for b in range(_NB):
                step(i + b, b, i + b + _LA, True)

        for e in range(n_epi):
            c = _NB + n_loop + e
            rc = c + _LA
            step(c, c % _NB, rc if rc < H else None, True)
        for c in range(H - _NB, H):
            wait_write(c % _NB)

    return k


def kernel(seqs, table):
    B, H = seqs.shape
    V, D = table.shape
    assert B % (_NW * 8) == 0
    assert H >= _NB + _LA
    idx_t = seqs.astype(jnp.int32).T
    out_t = _build(V, D, B, H)(idx_t, table)
    return jnp.transpose(out_t, (1, 0, 2))
